# baseline (speedup 1.0000x reference)
import jax
import jax.numpy as jnp
from jax.experimental import pallas as pl
from jax.experimental.pallas import tpu as pltpu
from jax.experimental.pallas import tpu_sc as plsc

N = 4096
R = 4
INDIM = 128
OUTDIM = 16

BM = 128
NTC = 4
ROWS_TC = N // NTC
SM = 512
SB = 2


def _xw_kernel(x_ref, w2_ref, xw_ref):
    y = jnp.dot(x_ref[...], w2_ref[...], preferred_element_type=jnp.float32)
    for r in range(R):
        xw_ref[r * N:(r + 1) * N, :] = (
            y[:, r * OUTDIM:(r + 1) * OUTDIM].astype(jnp.bfloat16))


def _agg_kernel(xw_ref, a_ref, o_ref):
    acc = jnp.dot(a_ref[...].astype(jnp.bfloat16), xw_ref[...],
                  preferred_element_type=jnp.float32)
    o_ref[...] = jnp.maximum(acc, 0.0)


def _tc_parts(X, A, W):
    W2 = W.reshape(R, INDIM, OUTDIM).transpose(1, 0, 2).reshape(
        INDIM, R * OUTDIM)
    xw = pl.pallas_call(
        _xw_kernel,
        in_specs=[pl.BlockSpec(memory_space=pltpu.VMEM),
                  pl.BlockSpec(memory_space=pltpu.VMEM)],
        out_specs=pl.BlockSpec(memory_space=pltpu.VMEM),
        out_shape=jax.ShapeDtypeStruct((R * N, OUTDIM), jnp.bfloat16),
    )(X, W2)
    outs = []
    for t in range(NTC):
        outs.append(pl.pallas_call(
            _agg_kernel,
            grid=(ROWS_TC // BM,),
            in_specs=[
                pl.BlockSpec((R * N, OUTDIM), lambda m: (0, 0)),
                pl.BlockSpec((BM, R * N),
                             lambda m, t=t: (t * (ROWS_TC // BM) + m, 0)),
            ],
            out_specs=pl.BlockSpec((BM, OUTDIM), lambda m: (m, 0)),
            out_shape=jax.ShapeDtypeStruct((ROWS_TC, OUTDIM), jnp.float32),
        )(xw, A))
    return jnp.concatenate(outs, axis=0)


def _sc_part(A):
    vector_mesh = plsc.VectorSubcoreMesh(
        core_axis_name="core", subcore_axis_name="subcore")

    @pl.kernel(out_type=jax.ShapeDtypeStruct((SM, OUTDIM), jnp.float32),
               mesh=vector_mesh)
    def sc_probe(a_hbm, o_hbm):
        def body(a_vmem, o_vmem):
            for r in range(SB):
                o_vmem[r:r + 1, :] = a_vmem[r:r + 1, :OUTDIM]

        pltpu.emit_pipeline(
            body,
            grid=(SM // SB,),
            in_specs=[pl.BlockSpec((SB, R * N),
                                   index_map=lambda i: (i + (N - SM) // SB, 0))],
            out_specs=[pl.BlockSpec((SB, OUTDIM), index_map=lambda i: (i, 0))],
            core_axis_name=("core", "subcore"),
            dimension_semantics=(pltpu.PARALLEL,),
        )(a_hbm, o_hbm)

    return sc_probe(A)


def kernel(X, A, W):
    sc_out = _sc_part(A)
    out_tc = _tc_parts(X, A, W)
    return out_tc.at[N - SM:, :].add(0.0 * sc_out)


# speedup vs baseline: 1.4904x; 1.4904x over previous
import jax
import jax.numpy as jnp
from jax.experimental import pallas as pl
from jax.experimental.pallas import tpu as pltpu

N = 4096
R = 4
INDIM = 128
OUTDIM = 16
PADO = 128

BM = 128


def _mrgcn_kernel(x_ref, w2_ref, a_ref, o_ref, xw_ref):
    @pl.when(pl.program_id(0) == 0)
    def _():
        y = jnp.dot(x_ref[...], w2_ref[...],
                    preferred_element_type=jnp.float32)
        for r in range(R):
            xw_ref[r * N:(r + 1) * N, :] = y[:, r * PADO:(r + 1) * PADO].astype(
                jnp.bfloat16)

    acc = jnp.dot(a_ref[...].astype(jnp.bfloat16), xw_ref[...],
                  preferred_element_type=jnp.float32)
    o_ref[...] = jnp.maximum(acc, 0.0)


def kernel(X, A, W):
    Wv = W.reshape(R, INDIM, OUTDIM).transpose(1, 0, 2)
    W2 = jnp.pad(Wv, ((0, 0), (0, 0), (0, PADO - OUTDIM))).reshape(
        INDIM, R * PADO)
    out = pl.pallas_call(
        _mrgcn_kernel,
        grid=(N // BM,),
        in_specs=[
            pl.BlockSpec((N, INDIM), lambda m: (0, 0)),
            pl.BlockSpec((INDIM, R * PADO), lambda m: (0, 0)),
            pl.BlockSpec((BM, R * N), lambda m: (m, 0)),
        ],
        out_specs=pl.BlockSpec((BM, PADO), lambda m: (m, 0)),
        out_shape=jax.ShapeDtypeStruct((N, PADO), jnp.float32),
        scratch_shapes=[pltpu.VMEM((R * N, PADO), jnp.bfloat16)],
    )(X, W2, A)
    return out[:, :OUTDIM]
